# baseline (speedup 1.0000x reference)
import functools

import jax
import jax.numpy as jnp
from jax import lax
from jax.experimental import pallas as pl
from jax.experimental.pallas import tpu as pltpu
from jax.experimental.pallas import tpu_sc as plsc

_FEAT = 128
_BATCH = 16384
_NUM_WORKERS = 32
_ROWS_PER_WORKER = _BATCH // _NUM_WORKERS
_CHUNK = 32
_NCHUNK = _ROWS_PER_WORKER // _CHUNK
_NBUF = 2
_LANES = 16
_VECS_PER_ROW = _FEAT // _LANES
_LOSS_WEIGHT = 1.0

_mesh = plsc.VectorSubcoreMesh(core_axis_name="c", subcore_axis_name="s")


@functools.partial(
    pl.kernel,
    mesh=_mesh,
    out_type=jax.ShapeDtypeStruct((_NUM_WORKERS, _LANES), jnp.float32),
    compiler_params=pltpu.CompilerParams(needs_layout_passes=False),
    scratch_types=[
        pltpu.VMEM((_NCHUNK, _CHUNK), jnp.int32),
        pltpu.VMEM((_NBUF, _CHUNK, _FEAT), jnp.float32),
        pltpu.VMEM((_NBUF, _CHUNK, _FEAT), jnp.float32),
        pltpu.VMEM((_LANES,), jnp.float32),
        pltpu.SemaphoreType.DMA,
        pltpu.SemaphoreType.DMA,
        pltpu.SemaphoreType.DMA,
        pltpu.SemaphoreType.DMA,
    ],
)
def _center_loss_sc(x_hbm, idx_hbm, tab_hbm, out_hbm,
                    idx_v, xring, cring, obuf,
                    sem_x0, sem_x1, sem_c0, sem_c1):
    wid = lax.axis_index("s") * 2 + lax.axis_index("c")
    base = wid * _ROWS_PER_WORKER
    xsems = (sem_x0, sem_x1)
    csems = (sem_c0, sem_c1)

    xps = [None] * _NCHUNK
    cps = [None] * _NCHUNK
    for k in range(_NBUF):
        xps[k] = pltpu.async_copy(
            x_hbm.at[pl.ds(base + k * _CHUNK, _CHUNK)], xring.at[k],
            xsems[k])
    pltpu.sync_copy(idx_hbm.at[wid], idx_v)
    for k in range(_NBUF):
        cps[k] = pltpu.async_copy(tab_hbm.at[idx_v.at[k]], cring.at[k],
                                  csems[k])

    zero = jnp.zeros((_LANES,), jnp.float32)
    accs = (zero, zero, zero, zero)

    for k in range(_NCHUNK):
        xps[k].wait()
        cps[k].wait()
        xbuf = xring.at[k % _NBUF]
        cbuf = cring.at[k % _NBUF]

        @plsc.parallel_loop(0, _CHUNK, unroll=2, carry=accs)
        def row_body(r, acc, _xbuf=xbuf, _cbuf=cbuf):
            a0, a1, a2, a3 = acc
            for j in range(_VECS_PER_ROW):
                xv = _xbuf[r, pl.ds(j * _LANES, _LANES)]
                cv = _cbuf[r, pl.ds(j * _LANES, _LANES)]
                d = xv - cv
                if j % 4 == 0:
                    a0 = a0 + d * d
                elif j % 4 == 1:
                    a1 = a1 + d * d
                elif j % 4 == 2:
                    a2 = a2 + d * d
                else:
                    a3 = a3 + d * d
            return (a0, a1, a2, a3)

        accs = row_body
        if k + _NBUF < _NCHUNK:
            xps[k + _NBUF] = pltpu.async_copy(
                x_hbm.at[pl.ds(base + (k + _NBUF) * _CHUNK, _CHUNK)],
                xring.at[k % _NBUF], xsems[k % _NBUF])
            cps[k + _NBUF] = pltpu.async_copy(
                tab_hbm.at[idx_v.at[k + _NBUF]], cring.at[k % _NBUF],
                csems[k % _NBUF])

    obuf[...] = (accs[0] + accs[1]) + (accs[2] + accs[3])
    pltpu.sync_copy(obuf, out_hbm.at[wid])


def kernel(inputs, targets, centers):
    idx = targets.astype(jnp.int32).reshape(_NUM_WORKERS, _NCHUNK, _CHUNK)
    partials = _center_loss_sc(inputs, idx, centers)
    return jnp.sum(partials) * (_LOSS_WEIGHT / _BATCH)


# speedup vs baseline: 1.1330x; 1.1330x over previous
import functools

import jax
import jax.numpy as jnp
from jax import lax
from jax.experimental import pallas as pl
from jax.experimental.pallas import tpu as pltpu
from jax.experimental.pallas import tpu_sc as plsc

_FEAT = 128
_BATCH = 16384
_NUM_WORKERS = 32
_ROWS_PER_WORKER = _BATCH // _NUM_WORKERS
_CHUNK = 64
_NCHUNK = _ROWS_PER_WORKER // _CHUNK
_NBUF = 3
_LANES = 16
_VECS_PER_ROW = _FEAT // _LANES
_LOSS_WEIGHT = 1.0

_mesh = plsc.VectorSubcoreMesh(core_axis_name="c", subcore_axis_name="s")


@functools.partial(
    pl.kernel,
    mesh=_mesh,
    out_type=jax.ShapeDtypeStruct((_NUM_WORKERS, _LANES), jnp.float32),
    compiler_params=pltpu.CompilerParams(needs_layout_passes=False),
    scratch_types=[
        pltpu.VMEM((_NCHUNK, _CHUNK), jnp.int32),
        pltpu.VMEM((_NBUF, _CHUNK, _FEAT), jnp.float32),
        pltpu.VMEM((_NBUF, _CHUNK, _FEAT), jnp.float32),
        pltpu.VMEM((_LANES,), jnp.float32),
        pltpu.SemaphoreType.DMA,
        pltpu.SemaphoreType.DMA,
        pltpu.SemaphoreType.DMA,
        pltpu.SemaphoreType.DMA,
        pltpu.SemaphoreType.DMA,
        pltpu.SemaphoreType.DMA,
    ],
)
def _center_loss_sc(x_hbm, idx_hbm, tab_hbm, out_hbm,
                    idx_v, xring, cring, obuf,
                    sem_x0, sem_x1, sem_x2, sem_c0, sem_c1, sem_c2):
    wid = lax.axis_index("s") * 2 + lax.axis_index("c")
    base = wid * _ROWS_PER_WORKER
    xsems = (sem_x0, sem_x1, sem_x2)
    csems = (sem_c0, sem_c1, sem_c2)

    xps = [None] * _NCHUNK
    cps = [None] * _NCHUNK
    for k in range(_NBUF):
        xps[k] = pltpu.async_copy(
            x_hbm.at[pl.ds(base + k * _CHUNK, _CHUNK)], xring.at[k],
            xsems[k])
    pltpu.sync_copy(idx_hbm.at[wid], idx_v)
    for k in range(_NBUF):
        cps[k] = pltpu.async_copy(tab_hbm.at[idx_v.at[k]], cring.at[k],
                                  csems[k])

    zero = jnp.zeros((_LANES,), jnp.float32)
    accs = (zero, zero, zero, zero)

    for k in range(_NCHUNK):
        xps[k].wait()
        cps[k].wait()
        xbuf = xring.at[k % _NBUF]
        cbuf = cring.at[k % _NBUF]

        @plsc.parallel_loop(0, _CHUNK, unroll=2, carry=accs)
        def row_body(r, acc, _xbuf=xbuf, _cbuf=cbuf):
            a0, a1, a2, a3 = acc
            for j in range(_VECS_PER_ROW):
                xv = _xbuf[r, pl.ds(j * _LANES, _LANES)]
                cv = _cbuf[r, pl.ds(j * _LANES, _LANES)]
                d = xv - cv
                if j % 4 == 0:
                    a0 = a0 + d * d
                elif j % 4 == 1:
                    a1 = a1 + d * d
                elif j % 4 == 2:
                    a2 = a2 + d * d
                else:
                    a3 = a3 + d * d
            return (a0, a1, a2, a3)

        accs = row_body
        if k + _NBUF < _NCHUNK:
            xps[k + _NBUF] = pltpu.async_copy(
                x_hbm.at[pl.ds(base + (k + _NBUF) * _CHUNK, _CHUNK)],
                xring.at[k % _NBUF], xsems[k % _NBUF])
            cps[k + _NBUF] = pltpu.async_copy(
                tab_hbm.at[idx_v.at[k + _NBUF]], cring.at[k % _NBUF],
                csems[k % _NBUF])

    obuf[...] = (accs[0] + accs[1]) + (accs[2] + accs[3])
    pltpu.sync_copy(obuf, out_hbm.at[wid])


def kernel(inputs, targets, centers):
    idx = targets.astype(jnp.int32).reshape(_NUM_WORKERS, _NCHUNK, _CHUNK)
    partials = _center_loss_sc(inputs, idx, centers)
    return jnp.sum(partials) * (_LOSS_WEIGHT / _BATCH)
